# baseline (speedup 1.0000x reference)
import functools

import jax
import jax.numpy as jnp
from jax import lax
from jax.experimental import pallas as pl
from jax.experimental.pallas import tpu as pltpu
from jax.experimental.pallas import tpu_sc as plsc


def _tc_pre(x, W, W_out, asv, adv):
    n, ic = x.shape
    ho = W.shape[1]
    oc = W_out.shape[1]
    nh = ho // oc
    blk = 400
    grid = n // blk

    def body(x_ref, w_ref, wo_ref, as_ref, ad_ref, p_ref, st_ref, dt_ref):
        xb = x_ref[...]
        h = jnp.dot(xb, w_ref[...], preferred_element_type=jnp.float32)
        h3 = h.reshape(blk, nh, oc)
        a_s = jnp.sum(h3 * as_ref[...][None], axis=-1)
        a_d = jnp.sum(h3 * ad_ref[...][None], axis=-1)
        st_ref[...] = jnp.concatenate([a_s, a_s], axis=1)
        dt_ref[...] = jnp.concatenate([a_d, a_d], axis=1)
        hc = oc // 2
        for hh in range(nh):
            ph = jnp.dot(
                h[:, hh * oc:(hh + 1) * oc], wo_ref[hh * oc:(hh + 1) * oc, :],
                preferred_element_type=jnp.float32)
            p_ref[0, :, hh * hc:(hh + 1) * hc] = ph[:, :hc]
            p_ref[1, :, hh * hc:(hh + 1) * hc] = ph[:, hc:]

    return pl.pallas_call(
        body,
        grid=(grid,),
        in_specs=[
            pl.BlockSpec((blk, ic), lambda i: (i, 0)),
            pl.BlockSpec((ic, ho), lambda i: (0, 0)),
            pl.BlockSpec((ho, oc), lambda i: (0, 0)),
            pl.BlockSpec((nh, oc), lambda i: (0, 0)),
            pl.BlockSpec((nh, oc), lambda i: (0, 0)),
        ],
        out_specs=[
            pl.BlockSpec((2, blk, ho // 2), lambda i: (0, i, 0)),
            pl.BlockSpec((blk, 2 * nh), lambda i: (i, 0)),
            pl.BlockSpec((blk, 2 * nh), lambda i: (i, 0)),
        ],
        out_shape=[
            jax.ShapeDtypeStruct((2, n, ho // 2), jnp.float32),
            jax.ShapeDtypeStruct((n, 2 * nh), jnp.float32),
            jax.ShapeDtypeStruct((n, 2 * nh), jnp.float32),
        ],
    )(x, W, W_out, asv, adv)


def _sc_agg(ast, adt, esrc2r, edstr, p2):
    n = ast.shape[0]
    lanes = ast.shape[1]
    nh = lanes // 2
    hf = p2.shape[1]
    ohc = hf // nh
    ch = edstr.shape[1]
    nrows = edstr.shape[0]
    e = nrows * ch
    info = plsc.get_sparse_core_info()
    nc, ns = info.num_cores, info.num_subcores
    ncks = nrows // ns
    sup = 100
    nsup = ncks // sup
    nrc = n // ch
    per = -(-nrc // ns)
    mesh = plsc.VectorSubcoreMesh(core_axis_name="c", subcore_axis_name="s")

    @functools.partial(
        pl.kernel,
        out_type=jax.ShapeDtypeStruct((nc, n, ohc), jnp.float32),
        mesh=mesh,
        compiler_params=pltpu.CompilerParams(use_tc_tiling_on_sc=False),
        scratch_types=[
            pltpu.VMEM((sup, ch), jnp.int32),
            pltpu.VMEM((sup, ch), jnp.int32),
            pltpu.VMEM((sup, ch), jnp.int32),
            [pltpu.VMEM((ch, lanes), jnp.float32)] * 2,
            [pltpu.VMEM((ch, lanes), jnp.float32)] * 2,
            [pltpu.VMEM((ch, lanes), jnp.float32)] * 2,
            [pltpu.VMEM((ch, hf), jnp.float32)] * 2,
            pltpu.VMEM((ch, lanes), jnp.float32),
            [pltpu.VMEM((ch, ohc), jnp.float32)] * 2,
            pltpu.VMEM((ch, ohc), jnp.float32),
            pltpu.VMEM_SHARED((n, lanes), jnp.float32),
            pltpu.VMEM_SHARED((n, ohc), jnp.float32),
            [[pltpu.SemaphoreType.DMA] * 5] * 2,
        ],
    )
    def k(ast_ref, adt_ref, esrc_ref, edst_ref, p_ref, out_ref,
          sup_s, sup_p, sup_d, srows, drows, denrows, prows, wrows, mbuf, zbuf,
          den_sh, acc_sh, sem):
        cid = lax.axis_index("c")
        sid = lax.axis_index("s")

        def zrow(r, _):
            for j in range(ohc // 16):
                z16 = jnp.zeros((16,), jnp.float32)
                mbuf[0][r, pl.ds(16 * j, 16)] = z16
                mbuf[1][r, pl.ds(16 * j, 16)] = z16
                zbuf[r, pl.ds(16 * j, 16)] = z16
            wrows[r, :] = jnp.zeros((lanes,), jnp.float32)
            return 0
        lax.fori_loop(0, ch, zrow, 0)
        for kk in range(per):
            cix = sid + ns * kk
            @pl.when(cix < nrc)
            def _():
                base = pl.multiple_of(cix * ch, 8)
                pltpu.sync_copy(zbuf, acc_sh.at[pl.ds(base, ch), :])
                pltpu.sync_copy(wrows, den_sh.at[pl.ds(base, ch), :])
        plsc.subcore_barrier()

        def a_start(sl, kk):
            pltpu.async_copy(ast_ref.at[sup_s.at[kk]], srows[sl], sem[sl][0])
            pltpu.async_copy(adt_ref.at[sup_d.at[kk]], drows[sl], sem[sl][1])

        def a_finish(sl, kk):
            pltpu.make_async_copy(ast_ref.at[sup_s.at[kk]], srows[sl], sem[sl][0]).wait()
            pltpu.make_async_copy(adt_ref.at[sup_d.at[kk]], drows[sl], sem[sl][1]).wait()
            for i in range(ch):
                v = srows[sl][i, :] + drows[sl][i, :]
                v = jnp.where(v >= 0.0, v, 0.2 * v)
                wrows[i, :] = jnp.exp(v)
            pltpu.sync_copy(wrows, den_sh.at[sup_d.at[kk]], add=True)

        def super_a(s, _):
            row = sid * ncks + s * sup
            pltpu.sync_copy(esrc_ref.at[pl.ds(row, sup), :], sup_s)
            pltpu.sync_copy(edst_ref.at[pl.ds(row, sup), :], sup_d)
            a_start(0, 0)
            a_start(1, 1)
            def pair(g, _):
                a_finish(0, 2 * g)
                a_start(0, 2 * g + 2)
                a_finish(1, 2 * g + 1)
                a_start(1, 2 * g + 3)
                return 0
            lax.fori_loop(0, sup // 2 - 1, pair, 0)
            a_finish(0, sup - 2)
            a_finish(1, sup - 1)
            return 0
        lax.fori_loop(0, nsup, super_a, 0)
        plsc.subcore_barrier()

        def b_start(sl, kk):
            pltpu.async_copy(ast_ref.at[sup_s.at[kk]], srows[sl], sem[sl][0])
            pltpu.async_copy(adt_ref.at[sup_d.at[kk]], drows[sl], sem[sl][1])
            pltpu.async_copy(den_sh.at[sup_d.at[kk]], denrows[sl], sem[sl][2])
            pltpu.async_copy(p_ref.at[sup_p.at[kk]], prows[sl], sem[sl][3])

        def b_finish(sl, kk):
            pltpu.make_async_copy(ast_ref.at[sup_s.at[kk]], srows[sl], sem[sl][0]).wait()
            pltpu.make_async_copy(adt_ref.at[sup_d.at[kk]], drows[sl], sem[sl][1]).wait()
            pltpu.make_async_copy(den_sh.at[sup_d.at[kk]], denrows[sl], sem[sl][2]).wait()
            pltpu.make_async_copy(p_ref.at[sup_p.at[kk]], prows[sl], sem[sl][3]).wait()
            for i in range(ch):
                v = srows[sl][i, :] + drows[sl][i, :]
                v = jnp.where(v >= 0.0, v, 0.2 * v)
                w = jnp.exp(v)
                wrows[i, :] = w / (denrows[sl][i, :] + 1e-16)
            pltpu.make_async_copy(mbuf[sl], acc_sh.at[sup_d.at[kk]], sem[sl][4]).wait()
            def medge(i, _):
                arow = wrows[i, :]
                for j in range(ohc // 16):
                    acc = jnp.zeros((16,), jnp.float32)
                    for hh in range(nh):
                        acc = acc + arow[hh] * prows[sl][i, pl.ds(hh * ohc + j * 16, 16)]
                    mbuf[sl][i, pl.ds(j * 16, 16)] = acc
                return 0
            lax.fori_loop(0, ch, medge, 0, unroll=4)
            pltpu.async_copy(mbuf[sl], acc_sh.at[sup_d.at[kk]], sem[sl][4], add=True)

        def super_b(s, _):
            row = sid * ncks + s * sup
            pltpu.sync_copy(esrc_ref.at[pl.ds(row, sup), :], sup_s)
            pltpu.sync_copy(esrc_ref.at[pl.ds(cid * nrows + row, sup), :], sup_p)
            pltpu.sync_copy(edst_ref.at[pl.ds(row, sup), :], sup_d)
            pltpu.async_copy(zbuf, acc_sh.at[sup_d.at[0]], sem[0][4], add=True)
            pltpu.async_copy(zbuf, acc_sh.at[sup_d.at[1]], sem[1][4], add=True)
            b_start(0, 0)
            b_start(1, 1)
            def pair(g, _):
                b_finish(0, 2 * g)
                b_start(0, 2 * g + 2)
                b_finish(1, 2 * g + 1)
                b_start(1, 2 * g + 3)
                return 0
            lax.fori_loop(0, sup // 2 - 1, pair, 0)
            b_finish(0, sup - 2)
            b_finish(1, sup - 1)
            pltpu.make_async_copy(mbuf[0], acc_sh.at[sup_d.at[0]], sem[0][4]).wait()
            pltpu.make_async_copy(mbuf[1], acc_sh.at[sup_d.at[1]], sem[1][4]).wait()
            return 0
        lax.fori_loop(0, nsup, super_b, 0)
        plsc.subcore_barrier()

        for kk in range(per):
            cix = sid + ns * kk
            @pl.when(cix < nrc)
            def _():
                base = pl.multiple_of(cix * ch, 8)
                pltpu.sync_copy(acc_sh.at[pl.ds(base, ch), :], mbuf[0])
                pltpu.sync_copy(mbuf[0], out_ref.at[cid, pl.ds(base, ch), :])

    return k(ast, adt, esrc2r, edstr, p2)


def _combine(acc2, bias_gat, W_out, b_out):
    nc, n, ohc = acc2.shape
    ho = W_out.shape[0]
    blk = 400
    grid = n // blk

    def body(a_ref, bg_ref, wo_ref, bo_ref, o_ref):
        bc = jnp.dot(bg_ref[...], wo_ref[...],
                     preferred_element_type=jnp.float32) + bo_ref[...]
        o_ref[...] = jnp.concatenate([a_ref[0], a_ref[1]], axis=1) + bc

    oc = 2 * ohc
    return pl.pallas_call(
        body,
        grid=(grid,),
        in_specs=[
            pl.BlockSpec((nc, blk, ohc), lambda i: (0, i, 0)),
            pl.BlockSpec((1, ho), lambda i: (0, 0)),
            pl.BlockSpec((ho, oc), lambda i: (0, 0)),
            pl.BlockSpec((1, oc), lambda i: (0, 0)),
        ],
        out_specs=pl.BlockSpec((blk, oc), lambda i: (i, 0)),
        out_shape=jax.ShapeDtypeStruct((n, oc), jnp.float32),
    )(acc2, bias_gat.reshape(1, ho), W_out, b_out.reshape(1, oc))


def kernel(x, edge_index, W, att_src, att_dst, bias_gat, W_out, b_out):
    ho = W.shape[1]
    oc = W_out.shape[1]
    nh = ho // oc
    esrc = edge_index[0]
    edst = edge_index[1]
    n = x.shape[0]
    ch = 40
    esrc2r = jnp.concatenate([esrc, esrc + n]).reshape(-1, ch)
    edstr = edst.reshape(-1, ch)
    asv = att_src.reshape(nh, oc)
    adv = att_dst.reshape(nh, oc)
    p, ast, adt = _tc_pre(x, W, W_out, asv, adv)
    p2 = p.reshape(2 * p.shape[1], p.shape[2])
    acc2 = _sc_agg(ast, adt, esrc2r, edstr, p2)
    return _combine(acc2, bias_gat, W_out, b_out)


# speedup vs baseline: 1.0131x; 1.0131x over previous
import functools

import jax
import jax.numpy as jnp
from jax import lax
from jax.experimental import pallas as pl
from jax.experimental.pallas import tpu as pltpu
from jax.experimental.pallas import tpu_sc as plsc


def _tc_pre(x, W, W_out, asv, adv):
    n, ic = x.shape
    ho = W.shape[1]
    oc = W_out.shape[1]
    nh = ho // oc
    blk = 400
    grid = n // blk

    def body(x_ref, w_ref, wo_ref, as_ref, ad_ref, p_ref, st_ref, dt_ref):
        xb = x_ref[...]
        h = jnp.dot(xb, w_ref[...], preferred_element_type=jnp.float32)
        h3 = h.reshape(blk, nh, oc)
        a_s = jnp.sum(h3 * as_ref[...][None], axis=-1)
        a_d = jnp.sum(h3 * ad_ref[...][None], axis=-1)
        st_ref[...] = jnp.concatenate([a_s, a_s], axis=1)
        dt_ref[...] = jnp.concatenate([a_d, a_d], axis=1)
        hc = oc // 2
        for hh in range(nh):
            ph = jnp.dot(
                h[:, hh * oc:(hh + 1) * oc], wo_ref[hh * oc:(hh + 1) * oc, :],
                preferred_element_type=jnp.float32)
            p_ref[0, :, hh * hc:(hh + 1) * hc] = ph[:, :hc]
            p_ref[1, :, hh * hc:(hh + 1) * hc] = ph[:, hc:]

    return pl.pallas_call(
        body,
        grid=(grid,),
        in_specs=[
            pl.BlockSpec((blk, ic), lambda i: (i, 0)),
            pl.BlockSpec((ic, ho), lambda i: (0, 0)),
            pl.BlockSpec((ho, oc), lambda i: (0, 0)),
            pl.BlockSpec((nh, oc), lambda i: (0, 0)),
            pl.BlockSpec((nh, oc), lambda i: (0, 0)),
        ],
        out_specs=[
            pl.BlockSpec((2, blk, ho // 2), lambda i: (0, i, 0)),
            pl.BlockSpec((blk, 2 * nh), lambda i: (i, 0)),
            pl.BlockSpec((blk, 2 * nh), lambda i: (i, 0)),
        ],
        out_shape=[
            jax.ShapeDtypeStruct((2, n, ho // 2), jnp.float32),
            jax.ShapeDtypeStruct((n, 2 * nh), jnp.float32),
            jax.ShapeDtypeStruct((n, 2 * nh), jnp.float32),
        ],
    )(x, W, W_out, asv, adv)


def _sc_agg(ast, adt, esrc2r, edstr, p2):
    n = ast.shape[0]
    lanes = ast.shape[1]
    nh = lanes // 2
    hf = p2.shape[1]
    ohc = hf // nh
    ch = edstr.shape[1]
    nrows = edstr.shape[0]
    e = nrows * ch
    info = plsc.get_sparse_core_info()
    nc, ns = info.num_cores, info.num_subcores
    ncks = nrows // ns
    sup = 100
    nsup = ncks // sup
    nrc = n // ch
    per = -(-nrc // ns)
    mesh = plsc.VectorSubcoreMesh(core_axis_name="c", subcore_axis_name="s")

    @functools.partial(
        pl.kernel,
        out_type=jax.ShapeDtypeStruct((nc, n, ohc), jnp.float32),
        mesh=mesh,
        compiler_params=pltpu.CompilerParams(use_tc_tiling_on_sc=False),
        scratch_types=[
            pltpu.VMEM((sup, ch), jnp.int32),
            pltpu.VMEM((sup, ch), jnp.int32),
            pltpu.VMEM((sup, ch), jnp.int32),
            [pltpu.VMEM((ch, lanes), jnp.float32)] * 2,
            [pltpu.VMEM((ch, lanes), jnp.float32)] * 2,
            [pltpu.VMEM((ch, lanes), jnp.float32)] * 2,
            [pltpu.VMEM((ch, hf), jnp.float32)] * 2,
            [pltpu.VMEM((ch, lanes), jnp.float32)] * 2,
            [pltpu.VMEM((ch, ohc), jnp.float32)] * 2,
            pltpu.VMEM((ch, ohc), jnp.float32),
            pltpu.VMEM((ch, lanes), jnp.float32),
            pltpu.VMEM_SHARED((n, lanes), jnp.float32),
            pltpu.VMEM_SHARED((n, ohc), jnp.float32),
            [[pltpu.SemaphoreType.DMA] * 6] * 2,
        ],
    )
    def k(ast_ref, adt_ref, esrc_ref, edst_ref, p_ref, out_ref,
          sup_s, sup_p, sup_d, srows, drows, denrows, prows, wrows, mbuf, zbuf,
          zbuf16, den_sh, acc_sh, sem):
        cid = lax.axis_index("c")
        sid = lax.axis_index("s")

        def zrow(r, _):
            for j in range(ohc // 16):
                z16 = jnp.zeros((16,), jnp.float32)
                mbuf[0][r, pl.ds(16 * j, 16)] = z16
                mbuf[1][r, pl.ds(16 * j, 16)] = z16
                zbuf[r, pl.ds(16 * j, 16)] = z16
            z16l = jnp.zeros((lanes,), jnp.float32)
            wrows[0][r, :] = z16l
            wrows[1][r, :] = z16l
            zbuf16[r, :] = z16l
            return 0
        lax.fori_loop(0, ch, zrow, 0)
        for kk in range(per):
            cix = sid + ns * kk
            @pl.when(cix < nrc)
            def _():
                base = pl.multiple_of(cix * ch, 8)
                pltpu.sync_copy(zbuf, acc_sh.at[pl.ds(base, ch), :])
                pltpu.sync_copy(zbuf16, den_sh.at[pl.ds(base, ch), :])
        plsc.subcore_barrier()

        def a_start(sl, kk):
            pltpu.async_copy(ast_ref.at[sup_s.at[kk]], srows[sl], sem[sl][0])
            pltpu.async_copy(adt_ref.at[sup_d.at[kk]], drows[sl], sem[sl][1])

        def a_finish(sl, kk):
            pltpu.make_async_copy(ast_ref.at[sup_s.at[kk]], srows[sl], sem[sl][0]).wait()
            pltpu.make_async_copy(adt_ref.at[sup_d.at[kk]], drows[sl], sem[sl][1]).wait()
            pltpu.make_async_copy(wrows[sl], den_sh.at[sup_d.at[kk]], sem[sl][2]).wait()
            for i in range(ch):
                v = srows[sl][i, :] + drows[sl][i, :]
                v = jnp.where(v >= 0.0, v, 0.2 * v)
                wrows[sl][i, :] = jnp.exp(v)
            pltpu.async_copy(wrows[sl], den_sh.at[sup_d.at[kk]], sem[sl][2], add=True)

        def super_a(s, _):
            row = sid * ncks + s * sup
            pltpu.sync_copy(esrc_ref.at[pl.ds(row, sup), :], sup_s)
            pltpu.sync_copy(edst_ref.at[pl.ds(row, sup), :], sup_d)
            pltpu.async_copy(zbuf16, den_sh.at[sup_d.at[0]], sem[0][2], add=True)
            pltpu.async_copy(zbuf16, den_sh.at[sup_d.at[1]], sem[1][2], add=True)
            a_start(0, 0)
            a_start(1, 1)
            def pair(g, _):
                a_finish(0, 2 * g)
                a_start(0, 2 * g + 2)
                a_finish(1, 2 * g + 1)
                a_start(1, 2 * g + 3)
                return 0
            lax.fori_loop(0, sup // 2 - 1, pair, 0)
            a_finish(0, sup - 2)
            a_finish(1, sup - 1)
            pltpu.make_async_copy(wrows[0], den_sh.at[sup_d.at[0]], sem[0][2]).wait()
            pltpu.make_async_copy(wrows[1], den_sh.at[sup_d.at[1]], sem[1][2]).wait()
            return 0
        lax.fori_loop(0, nsup, super_a, 0)
        plsc.subcore_barrier()

        for kk in range(per):
            cix = sid + ns * kk
            @pl.when(cix < nrc)
            def _():
                base = pl.multiple_of(cix * ch, 8)
                pltpu.sync_copy(den_sh.at[pl.ds(base, ch), :], wrows[0])
                for i in range(ch):
                    wrows[1][i, :] = 1.0 / (wrows[0][i, :] + 1e-16)
                pltpu.sync_copy(wrows[1], den_sh.at[pl.ds(base, ch), :])
        plsc.subcore_barrier()

        def b_start(sl, kk):
            pltpu.async_copy(ast_ref.at[sup_s.at[kk]], srows[sl], sem[sl][0])
            pltpu.async_copy(adt_ref.at[sup_d.at[kk]], drows[sl], sem[sl][1])
            pltpu.async_copy(den_sh.at[sup_d.at[kk]], denrows[sl], sem[sl][2])
            h1 = 24
            pltpu.async_copy(p_ref.at[sup_p.at[kk, pl.ds(0, h1)]],
                             prows[sl].at[pl.ds(0, h1), :], sem[sl][3])
            pltpu.async_copy(p_ref.at[sup_p.at[kk, pl.ds(h1, ch - h1)]],
                             prows[sl].at[pl.ds(h1, ch - h1), :], sem[sl][5])

        def b_finish(sl, kk):
            pltpu.make_async_copy(ast_ref.at[sup_s.at[kk]], srows[sl], sem[sl][0]).wait()
            pltpu.make_async_copy(adt_ref.at[sup_d.at[kk]], drows[sl], sem[sl][1]).wait()
            pltpu.make_async_copy(den_sh.at[sup_d.at[kk]], denrows[sl], sem[sl][2]).wait()
            h1 = 24
            pltpu.make_async_copy(p_ref.at[sup_p.at[kk, pl.ds(0, h1)]],
                                  prows[sl].at[pl.ds(0, h1), :], sem[sl][3]).wait()
            pltpu.make_async_copy(p_ref.at[sup_p.at[kk, pl.ds(h1, ch - h1)]],
                                  prows[sl].at[pl.ds(h1, ch - h1), :], sem[sl][5]).wait()
            for i in range(ch):
                v = srows[sl][i, :] + drows[sl][i, :]
                v = jnp.where(v >= 0.0, v, 0.2 * v)
                w = jnp.exp(v)
                wrows[sl][i, :] = w * denrows[sl][i, :]
            pltpu.make_async_copy(mbuf[sl], acc_sh.at[sup_d.at[kk]], sem[sl][4]).wait()
            def medge(i, _):
                arow = wrows[sl][i, :]
                for j in range(ohc // 16):
                    acc = jnp.zeros((16,), jnp.float32)
                    for hh in range(nh):
                        acc = acc + arow[hh] * prows[sl][i, pl.ds(hh * ohc + j * 16, 16)]
                    mbuf[sl][i, pl.ds(j * 16, 16)] = acc
                return 0
            lax.fori_loop(0, ch, medge, 0, unroll=4)
            pltpu.async_copy(mbuf[sl], acc_sh.at[sup_d.at[kk]], sem[sl][4], add=True)

        def super_b(s, _):
            row = sid * ncks + s * sup
            pltpu.sync_copy(esrc_ref.at[pl.ds(row, sup), :], sup_s)
            pltpu.sync_copy(esrc_ref.at[pl.ds(cid * nrows + row, sup), :], sup_p)
            pltpu.sync_copy(edst_ref.at[pl.ds(row, sup), :], sup_d)
            pltpu.async_copy(zbuf, acc_sh.at[sup_d.at[0]], sem[0][4], add=True)
            pltpu.async_copy(zbuf, acc_sh.at[sup_d.at[1]], sem[1][4], add=True)
            b_start(0, 0)
            b_start(1, 1)
            def pair(g, _):
                b_finish(0, 2 * g)
                b_start(0, 2 * g + 2)
                b_finish(1, 2 * g + 1)
                b_start(1, 2 * g + 3)
                return 0
            lax.fori_loop(0, sup // 2 - 1, pair, 0)
            b_finish(0, sup - 2)
            b_finish(1, sup - 1)
            pltpu.make_async_copy(mbuf[0], acc_sh.at[sup_d.at[0]], sem[0][4]).wait()
            pltpu.make_async_copy(mbuf[1], acc_sh.at[sup_d.at[1]], sem[1][4]).wait()
            return 0
        lax.fori_loop(0, nsup, super_b, 0)
        plsc.subcore_barrier()

        for kk in range(per):
            cix = sid + ns * kk
            @pl.when(cix < nrc)
            def _():
                base = pl.multiple_of(cix * ch, 8)
                pltpu.sync_copy(acc_sh.at[pl.ds(base, ch), :], mbuf[0])
                pltpu.sync_copy(mbuf[0], out_ref.at[cid, pl.ds(base, ch), :])

    return k(ast, adt, esrc2r, edstr, p2)


def _combine(acc2, bias_gat, W_out, b_out):
    nc, n, ohc = acc2.shape
    ho = W_out.shape[0]
    blk = 400
    grid = n // blk

    def body(a_ref, bg_ref, wo_ref, bo_ref, o_ref):
        bc = jnp.dot(bg_ref[...], wo_ref[...],
                     preferred_element_type=jnp.float32) + bo_ref[...]
        o_ref[...] = jnp.concatenate([a_ref[0], a_ref[1]], axis=1) + bc

    oc = 2 * ohc
    return pl.pallas_call(
        body,
        grid=(grid,),
        in_specs=[
            pl.BlockSpec((nc, blk, ohc), lambda i: (0, i, 0)),
            pl.BlockSpec((1, ho), lambda i: (0, 0)),
            pl.BlockSpec((ho, oc), lambda i: (0, 0)),
            pl.BlockSpec((1, oc), lambda i: (0, 0)),
        ],
        out_specs=pl.BlockSpec((blk, oc), lambda i: (i, 0)),
        out_shape=jax.ShapeDtypeStruct((n, oc), jnp.float32),
    )(acc2, bias_gat.reshape(1, ho), W_out, b_out.reshape(1, oc))


def kernel(x, edge_index, W, att_src, att_dst, bias_gat, W_out, b_out):
    ho = W.shape[1]
    oc = W_out.shape[1]
    nh = ho // oc
    esrc = edge_index[0]
    edst = edge_index[1]
    n = x.shape[0]
    ch = 40
    esrc2r = jnp.concatenate([esrc, esrc + n]).reshape(-1, ch)
    edstr = edst.reshape(-1, ch)
    asv = att_src.reshape(nh, oc)
    adv = att_dst.reshape(nh, oc)
    p, ast, adt = _tc_pre(x, W, W_out, asv, adv)
    p2 = p.reshape(2 * p.shape[1], p.shape[2])
    acc2 = _sc_agg(ast, adt, esrc2r, edstr, p2)
    return _combine(acc2, bias_gat, W_out, b_out)
